# baseline (speedup 1.0000x reference)
import jax
import jax.numpy as jnp
from jax import lax
from jax.experimental import pallas as pl
from jax.experimental.pallas import tpu as pltpu
from jax.experimental.pallas import tpu_sc as plsc

_B, _V = 1024, 100000
_NC, _NS, _L = 2, 16, 16
_NW = _NC * _NS
_RSC = 512
_RTC = _B - _RSC
_RPW = _RSC // _NW
_TROWS = 64
_CW = 2560
_NCHUNK = 39
_XTRA0 = _CW * _NCHUNK
_TAIL0 = _XTRA0 + 128
_NQ = _CW // _L
_TPC = _CW // 128


def _k1_body(tgt_sm, *refs):
    tile_refs = refs[:8]
    tail_ref, t_ref, tailcnt_ref = refs[8:]
    i = pl.program_id(0)

    sub = lax.broadcasted_iota(jnp.int32, (8, 128), 0)
    ln = lax.broadcasted_iota(jnp.int32, (8, 128), 1)
    iota8 = lax.broadcasted_iota(jnp.int32, (8, 1), 0)

    t_vec = jnp.zeros((8, 1), jnp.float32)
    tgt_vec = jnp.zeros((8, 1), jnp.int32)
    for k in range(8):
        tgtk = tgt_sm[i * 8 + k]
        lane = tgtk - (tgtk // 128) * 128
        tile = tile_refs[k][...]
        t_k = jnp.max(jnp.where((sub == k) & (ln == lane), tile, -jnp.inf))
        t_vec += jnp.where(iota8 == k, t_k, 0.0)
        tgt_vec += jnp.where(iota8 == k, tgtk, 0)

    x_tail = tail_ref[...]
    col = _TAIL0 + lax.broadcasted_iota(jnp.int32, (8, 128), 1)
    valid = col < _V
    beats = ((x_tail > t_vec) | ((x_tail == t_vec) & (col < tgt_vec))) & valid
    tailcnt = jnp.sum(beats.astype(jnp.int32), axis=1, keepdims=True)

    t_ref[...] = t_vec
    tailcnt_ref[...] = tailcnt


def _k1(logits, tgt):
    def tile_spec(k):
        return pl.BlockSpec(
            (8, 128), lambda i, tgt_sm, k=k: (i, tgt_sm[i * 8 + k] // 128))

    grid_spec = pltpu.PrefetchScalarGridSpec(
        num_scalar_prefetch=1,
        grid=(_RSC // 8,),
        in_specs=[tile_spec(k) for k in range(8)] + [
            pl.BlockSpec((8, 128), lambda i, tgt_sm: (i, _TAIL0 // 128)),
        ],
        out_specs=[
            pl.BlockSpec((8, 1), lambda i, tgt_sm: (i, 0)),
            pl.BlockSpec((8, 1), lambda i, tgt_sm: (i, 0)),
        ],
    )
    return pl.pallas_call(
        _k1_body,
        grid_spec=grid_spec,
        out_shape=[
            jax.ShapeDtypeStruct((_RSC, 1), jnp.float32),
            jax.ShapeDtypeStruct((_RSC, 1), jnp.int32),
        ],
    )(tgt, *([logits] * 9))


def _sc_body(x_hbm, t_hbm, tgt_hbm, out_hbm,
             buf0, buf1, bufe, tstage, gstage, accbuf,
             sem0, sem1, sem2):
    wid = lax.axis_index("s") * _NC + lax.axis_index("c")
    row0 = wid * _RPW

    pltpu.async_copy(t_hbm.at[pl.ds(row0, _RPW)],
                     tstage.at[pl.ds(0, _RPW)], sem2).wait()
    pltpu.async_copy(tgt_hbm.at[pl.ds(row0, _RPW)],
                     gstage.at[pl.ds(0, _RPW)], sem2).wait()

    zero16 = jnp.zeros((_L,), jnp.int32)
    for r in range(_RPW):
        accbuf[pl.ds(r * _L, _L)] = zero16

    lane = lax.broadcasted_iota(jnp.int32, (_L,), 0)

    def count_tiles(buf, s, t_s, is_ge, lo, hi):
        def body(ti, accs):
            out = list(accs)
            for k in range(8):
                x = buf[ti, s, pl.ds(k * _L, _L)]
                m = (x >= t_s) if is_ge else (x > t_s)
                out[k % 4] = out[k % 4] + jnp.where(m, 1, 0)
            return tuple(out)

        accs = plsc.parallel_loop(lo, hi, unroll=2, carry=(zero16,) * 4)(body)
        return accs[0] + accs[1] + accs[2] + accs[3]

    def chunk_work(g, c0, buf, tpc, tgts, tsplats):
        for s in range(8):
            rl = g * 8 + s
            tgt_s = tgts[s]
            t_s = tsplats[s]
            off = tgt_s - c0
            cw = tpc * 128

            @pl.when(off >= cw)
            def _below():
                d = count_tiles(buf, s, t_s, True, 0, tpc)
                accbuf[pl.ds(rl * _L, _L)] = accbuf[pl.ds(rl * _L, _L)] + d

            @pl.when(off < 0)
            def _above():
                d = count_tiles(buf, s, t_s, False, 0, tpc)
                accbuf[pl.ds(rl * _L, _L)] = accbuf[pl.ds(rl * _L, _L)] + d

            @pl.when((off >= 0) & (off < cw))
            def _mixed():
                ti_m = off // 128
                k_m = (off // _L) - ti_m * 8
                lane_b = off - (off // _L) * _L
                d = count_tiles(buf, s, t_s, True, 0, ti_m)
                d = d + count_tiles(buf, s, t_s, False, ti_m + 1, tpc)
                for k in range(8):
                    x = buf[ti_m, s, pl.ds(k * _L, _L)]
                    ge1 = jnp.where(x >= t_s, 1, 0)
                    gt1 = jnp.where(x > t_s, 1, 0)
                    mix = jnp.where(lane < lane_b, ge1, gt1)
                    pick = jnp.where(
                        k < k_m, ge1, jnp.where(k > k_m, gt1, mix))
                    d = d + pick
                accbuf[pl.ds(rl * _L, _L)] = accbuf[pl.ds(rl * _L, _L)] + d

    def group(g, _):
        base = row0 + g * 8

        tgts = [gstage[pl.ds(g * 8 + s, _L)][0] for s in range(8)]
        tsplats = [jnp.broadcast_to(tstage[pl.ds(g * 8 + s, _L)][0], (_L,))
                   for s in range(8)]

        def start(c0, buf, sem):
            for ti in range(_TPC):
                pltpu.async_copy(
                    x_hbm.at[pl.ds(base, 8), pl.ds(c0 + ti * 128, 128)],
                    buf.at[ti], sem)

        def wait(buf, sem):
            for ti in range(_TPC):
                pltpu.make_async_copy(
                    x_hbm.at[pl.ds(base, 8), pl.ds(0, 128)],
                    buf.at[ti], sem).wait()

        start(0, buf0, sem0)
        pltpu.async_copy(
            x_hbm.at[pl.ds(base, 8), pl.ds(_XTRA0, 128)], bufe.at[0], sem2)

        def ring(i, _):
            cc = i * 2

            @pl.when(cc + 1 < _NCHUNK)
            def _():
                start((cc + 1) * _CW, buf1, sem1)

            wait(buf0, sem0)
            chunk_work(g, cc * _CW, buf0, _TPC, tgts, tsplats)

            @pl.when(cc + 2 < _NCHUNK)
            def _():
                start((cc + 2) * _CW, buf0, sem0)

            @pl.when(cc + 1 < _NCHUNK)
            def _():
                wait(buf1, sem1)
                chunk_work(g, (cc + 1) * _CW, buf1, _TPC, tgts, tsplats)

            return 0

        lax.fori_loop(0, (_NCHUNK + 1) // 2, ring, 0)

        pltpu.make_async_copy(
            x_hbm.at[pl.ds(base, 8), pl.ds(_XTRA0, 128)],
            bufe.at[0], sem2).wait()
        chunk_work(g, _XTRA0, bufe, 1, tgts, tsplats)
        return 0

    lax.fori_loop(0, _RPW // 8, group, 0)
    pltpu.sync_copy(accbuf, out_hbm.at[wid])


def _sc_count(logits, t, tgt):
    mesh = plsc.VectorSubcoreMesh(core_axis_name="c", subcore_axis_name="s")
    return pl.kernel(
        _sc_body,
        mesh=mesh,
        compiler_params=pltpu.CompilerParams(use_tc_tiling_on_sc=True),
        out_type=jax.ShapeDtypeStruct((_NW, _RPW * _L), jnp.int32),
        scratch_types=[
            pltpu.VMEM((_TPC, 8, 128), jnp.float32),
            pltpu.VMEM((_TPC, 8, 128), jnp.float32),
            pltpu.VMEM((1, 8, 128), jnp.float32),
            pltpu.VMEM((_RPW + _L,), jnp.float32),
            pltpu.VMEM((_RPW + _L,), jnp.int32),
            pltpu.VMEM((_RPW * _L,), jnp.int32),
            pltpu.SemaphoreType.DMA,
            pltpu.SemaphoreType.DMA,
            pltpu.SemaphoreType.DMA,
        ],
    )(logits, t, tgt)


def _tc_body(x_ref, tgt_ref, rank_ref):
    x = x_ref[...]
    tgt = tgt_ref[...]
    iota = jax.lax.broadcasted_iota(jnp.int32, x.shape, 1)
    t = jnp.max(jnp.where(iota == tgt, x, -jnp.inf), axis=1, keepdims=True)
    beats = (x > t) | ((x == t) & (iota < tgt))
    cnt = jnp.sum(beats.astype(jnp.float32), axis=1, keepdims=True)
    rank_ref[...] = cnt + 1.0


def _tc_count(logits, tgt2d):
    off = _RSC // _TROWS
    return pl.pallas_call(
        _tc_body,
        grid=(_RTC // _TROWS,),
        in_specs=[
            pl.BlockSpec((_TROWS, _V), lambda i: (i + off, 0)),
            pl.BlockSpec((_TROWS, 1), lambda i: (i + off, 0)),
        ],
        out_specs=pl.BlockSpec((_TROWS, 1), lambda i: (i, 0)),
        out_shape=jax.ShapeDtypeStruct((_RTC, 1), jnp.float32),
        compiler_params=pltpu.CompilerParams(
            dimension_semantics=("arbitrary",),
        ),
    )(logits, tgt2d)


def _k3_body(cntm_ref, tail_ref, rank_tc_ref, pm_ref, out_ref):
    cnt = jnp.sum(cntm_ref[...], axis=1, keepdims=True)
    rank_sc = (cnt + tail_ref[...]).astype(jnp.float32) + 1.0
    pm = pm_ref[...].astype(jnp.float32)
    rr_sc = pm[:_RSC, :] / rank_sc
    rr_tc = pm[_RSC:, :] / rank_tc_ref[...]
    mrr = (jnp.sum(rr_sc) + jnp.sum(rr_tc)) / jnp.sum(pm)
    out_ref[...] = jnp.full((1, 1), mrr, jnp.float32)


def kernel(logits, targets, padding_mask):
    tgt = targets.astype(jnp.int32)
    pm = padding_mask.astype(jnp.int32)

    t, tailcnt = _k1(logits, tgt)
    cnt_main = _sc_count(logits, t.reshape(_RSC), tgt)
    rank_tc = _tc_count(logits, tgt.reshape(_B, 1))

    out = pl.pallas_call(
        _k3_body,
        out_shape=jax.ShapeDtypeStruct((1, 1), jnp.float32),
    )(cnt_main.reshape(_RSC, _L),
      tailcnt,
      rank_tc,
      pm.reshape(_B, 1))
    return out.reshape(())


# speedup vs baseline: 1.1876x; 1.1876x over previous
import jax
import jax.numpy as jnp
from jax import lax
from jax.experimental import pallas as pl
from jax.experimental.pallas import tpu as pltpu
from jax.experimental.pallas import tpu_sc as plsc

_B, _V = 1024, 100000
_NC, _NS, _L = 2, 16, 16
_NW = _NC * _NS
_RSC = 256
_RTC = _B - _RSC
_RPW = _RSC // _NW
_TROWS = 64
_CW = 2560
_NCHUNK = 39
_XTRA0 = _CW * _NCHUNK
_TAIL0 = _XTRA0 + 128
_NQ = _CW // _L
_TPC = _CW // 128


def _k1_body(tgt_sm, *refs):
    tile_refs = refs[:8]
    tail_ref, t_ref, tailcnt_ref = refs[8:]
    i = pl.program_id(0)

    sub = lax.broadcasted_iota(jnp.int32, (8, 128), 0)
    ln = lax.broadcasted_iota(jnp.int32, (8, 128), 1)
    iota8 = lax.broadcasted_iota(jnp.int32, (8, 1), 0)

    t_vec = jnp.zeros((8, 1), jnp.float32)
    tgt_vec = jnp.zeros((8, 1), jnp.int32)
    for k in range(8):
        tgtk = tgt_sm[i * 8 + k]
        lane = tgtk - (tgtk // 128) * 128
        tile = tile_refs[k][...]
        t_k = jnp.max(jnp.where((sub == k) & (ln == lane), tile, -jnp.inf))
        t_vec += jnp.where(iota8 == k, t_k, 0.0)
        tgt_vec += jnp.where(iota8 == k, tgtk, 0)

    x_tail = tail_ref[...]
    col = _TAIL0 + lax.broadcasted_iota(jnp.int32, (8, 128), 1)
    valid = col < _V
    beats = ((x_tail > t_vec) | ((x_tail == t_vec) & (col < tgt_vec))) & valid
    tailcnt = jnp.sum(beats.astype(jnp.int32), axis=1, keepdims=True)

    t_ref[...] = t_vec
    tailcnt_ref[...] = tailcnt


def _k1(logits, tgt):
    def tile_spec(k):
        return pl.BlockSpec(
            (8, 128), lambda i, tgt_sm, k=k: (i, tgt_sm[i * 8 + k] // 128))

    grid_spec = pltpu.PrefetchScalarGridSpec(
        num_scalar_prefetch=1,
        grid=(_RSC // 8,),
        in_specs=[tile_spec(k) for k in range(8)] + [
            pl.BlockSpec((8, 128), lambda i, tgt_sm: (i, _TAIL0 // 128)),
        ],
        out_specs=[
            pl.BlockSpec((8, 1), lambda i, tgt_sm: (i, 0)),
            pl.BlockSpec((8, 1), lambda i, tgt_sm: (i, 0)),
        ],
    )
    return pl.pallas_call(
        _k1_body,
        grid_spec=grid_spec,
        out_shape=[
            jax.ShapeDtypeStruct((_RSC, 1), jnp.float32),
            jax.ShapeDtypeStruct((_RSC, 1), jnp.int32),
        ],
    )(tgt, *([logits] * 9))


def _sc_body(x_hbm, t_hbm, tgt_hbm, out_hbm,
             buf0, buf1, bufe, tstage, gstage, accbuf,
             sem0, sem1, sem2):
    wid = lax.axis_index("s") * _NC + lax.axis_index("c")
    row0 = wid * _RPW

    pltpu.async_copy(t_hbm.at[pl.ds(row0, _RPW)],
                     tstage.at[pl.ds(0, _RPW)], sem2).wait()
    pltpu.async_copy(tgt_hbm.at[pl.ds(row0, _RPW)],
                     gstage.at[pl.ds(0, _RPW)], sem2).wait()

    zero16 = jnp.zeros((_L,), jnp.int32)
    for r in range(_RPW):
        accbuf[pl.ds(r * _L, _L)] = zero16

    lane = lax.broadcasted_iota(jnp.int32, (_L,), 0)

    def count_tiles(buf, s, t_s, is_ge, lo, hi):
        def body(ti, accs):
            out = list(accs)
            for k in range(8):
                x = buf[ti, s, pl.ds(k * _L, _L)]
                m = (x >= t_s) if is_ge else (x > t_s)
                out[k % 4] = out[k % 4] + jnp.where(m, 1, 0)
            return tuple(out)

        accs = plsc.parallel_loop(lo, hi, unroll=2, carry=(zero16,) * 4)(body)
        return accs[0] + accs[1] + accs[2] + accs[3]

    def chunk_work(g, c0, buf, tpc, tgts, tsplats):
        for s in range(8):
            rl = g * 8 + s
            tgt_s = tgts[s]
            t_s = tsplats[s]
            off = tgt_s - c0
            cw = tpc * 128

            @pl.when(off >= cw)
            def _below():
                d = count_tiles(buf, s, t_s, True, 0, tpc)
                accbuf[pl.ds(rl * _L, _L)] = accbuf[pl.ds(rl * _L, _L)] + d

            @pl.when(off < 0)
            def _above():
                d = count_tiles(buf, s, t_s, False, 0, tpc)
                accbuf[pl.ds(rl * _L, _L)] = accbuf[pl.ds(rl * _L, _L)] + d

            @pl.when((off >= 0) & (off < cw))
            def _mixed():
                ti_m = off // 128
                k_m = (off // _L) - ti_m * 8
                lane_b = off - (off // _L) * _L
                d = count_tiles(buf, s, t_s, True, 0, ti_m)
                d = d + count_tiles(buf, s, t_s, False, ti_m + 1, tpc)
                for k in range(8):
                    x = buf[ti_m, s, pl.ds(k * _L, _L)]
                    ge1 = jnp.where(x >= t_s, 1, 0)
                    gt1 = jnp.where(x > t_s, 1, 0)
                    mix = jnp.where(lane < lane_b, ge1, gt1)
                    pick = jnp.where(
                        k < k_m, ge1, jnp.where(k > k_m, gt1, mix))
                    d = d + pick
                accbuf[pl.ds(rl * _L, _L)] = accbuf[pl.ds(rl * _L, _L)] + d

    def group(g, _):
        base = row0 + g * 8

        tgts = [gstage[pl.ds(g * 8 + s, _L)][0] for s in range(8)]
        tsplats = [jnp.broadcast_to(tstage[pl.ds(g * 8 + s, _L)][0], (_L,))
                   for s in range(8)]

        def start(c0, buf, sem):
            for ti in range(_TPC):
                pltpu.async_copy(
                    x_hbm.at[pl.ds(base, 8), pl.ds(c0 + ti * 128, 128)],
                    buf.at[ti], sem)

        def wait(buf, sem):
            for ti in range(_TPC):
                pltpu.make_async_copy(
                    x_hbm.at[pl.ds(base, 8), pl.ds(0, 128)],
                    buf.at[ti], sem).wait()

        start(0, buf0, sem0)
        pltpu.async_copy(
            x_hbm.at[pl.ds(base, 8), pl.ds(_XTRA0, 128)], bufe.at[0], sem2)

        def ring(i, _):
            cc = i * 2

            @pl.when(cc + 1 < _NCHUNK)
            def _():
                start((cc + 1) * _CW, buf1, sem1)

            wait(buf0, sem0)
            chunk_work(g, cc * _CW, buf0, _TPC, tgts, tsplats)

            @pl.when(cc + 2 < _NCHUNK)
            def _():
                start((cc + 2) * _CW, buf0, sem0)

            @pl.when(cc + 1 < _NCHUNK)
            def _():
                wait(buf1, sem1)
                chunk_work(g, (cc + 1) * _CW, buf1, _TPC, tgts, tsplats)

            return 0

        lax.fori_loop(0, (_NCHUNK + 1) // 2, ring, 0)

        pltpu.make_async_copy(
            x_hbm.at[pl.ds(base, 8), pl.ds(_XTRA0, 128)],
            bufe.at[0], sem2).wait()
        chunk_work(g, _XTRA0, bufe, 1, tgts, tsplats)
        return 0

    lax.fori_loop(0, _RPW // 8, group, 0)
    pltpu.sync_copy(accbuf, out_hbm.at[wid])


def _sc_count(logits, t, tgt):
    mesh = plsc.VectorSubcoreMesh(core_axis_name="c", subcore_axis_name="s")
    return pl.kernel(
        _sc_body,
        mesh=mesh,
        compiler_params=pltpu.CompilerParams(use_tc_tiling_on_sc=True),
        out_type=jax.ShapeDtypeStruct((_NW, _RPW * _L), jnp.int32),
        scratch_types=[
            pltpu.VMEM((_TPC, 8, 128), jnp.float32),
            pltpu.VMEM((_TPC, 8, 128), jnp.float32),
            pltpu.VMEM((1, 8, 128), jnp.float32),
            pltpu.VMEM((_RPW + _L,), jnp.float32),
            pltpu.VMEM((_RPW + _L,), jnp.int32),
            pltpu.VMEM((_RPW * _L,), jnp.int32),
            pltpu.SemaphoreType.DMA,
            pltpu.SemaphoreType.DMA,
            pltpu.SemaphoreType.DMA,
        ],
    )(logits, t, tgt)


def _tc_body(x_ref, tgt_ref, rank_ref):
    x = x_ref[...]
    tgt = tgt_ref[...]
    iota = jax.lax.broadcasted_iota(jnp.int32, x.shape, 1)
    t = jnp.max(jnp.where(iota == tgt, x, -jnp.inf), axis=1, keepdims=True)
    beats = (x > t) | ((x == t) & (iota < tgt))
    cnt = jnp.sum(beats.astype(jnp.float32), axis=1, keepdims=True)
    rank_ref[...] = cnt + 1.0


def _tc_count(logits, tgt2d):
    off = _RSC // _TROWS
    return pl.pallas_call(
        _tc_body,
        grid=(_RTC // _TROWS,),
        in_specs=[
            pl.BlockSpec((_TROWS, _V), lambda i: (i + off, 0)),
            pl.BlockSpec((_TROWS, 1), lambda i: (i + off, 0)),
        ],
        out_specs=pl.BlockSpec((_TROWS, 1), lambda i: (i, 0)),
        out_shape=jax.ShapeDtypeStruct((_RTC, 1), jnp.float32),
        compiler_params=pltpu.CompilerParams(
            dimension_semantics=("arbitrary",),
        ),
    )(logits, tgt2d)


def _k3_body(cntm_ref, tail_ref, rank_tc_ref, pm_ref, out_ref):
    cnt = jnp.sum(cntm_ref[...], axis=1, keepdims=True)
    rank_sc = (cnt + tail_ref[...]).astype(jnp.float32) + 1.0
    pm = pm_ref[...].astype(jnp.float32)
    rr_sc = pm[:_RSC, :] / rank_sc
    rr_tc = pm[_RSC:, :] / rank_tc_ref[...]
    mrr = (jnp.sum(rr_sc) + jnp.sum(rr_tc)) / jnp.sum(pm)
    out_ref[...] = jnp.full((1, 1), mrr, jnp.float32)


def kernel(logits, targets, padding_mask):
    tgt = targets.astype(jnp.int32)
    pm = padding_mask.astype(jnp.int32)

    t, tailcnt = _k1(logits, tgt)
    cnt_main = _sc_count(logits, t.reshape(_RSC), tgt)
    rank_tc = _tc_count(logits, tgt.reshape(_B, 1))

    out = pl.pallas_call(
        _k3_body,
        out_shape=jax.ShapeDtypeStruct((1, 1), jnp.float32),
    )(cnt_main.reshape(_RSC, _L),
      tailcnt,
      rank_tc,
      pm.reshape(_B, 1))
    return out.reshape(())
